# Initial kernel scaffold; baseline (speedup 1.0000x reference)
import jax
import jax.numpy as jnp
from jax.experimental import pallas as pl


def kernel(x, edge_index, pos, W1, b1, W2, b2, Wout, bout):
    raise NotImplementedError("write your pallas kernel here")


# speedup vs baseline: 6.8007x; 6.8007x over previous
import functools

import jax
import jax.numpy as jnp
from jax import lax
from jax.experimental import pallas as pl
from jax.experimental.pallas import tpu as pltpu
from jax.experimental.pallas import tpu_sc as plsc

N_NODES = 10000
N_EDGES = 320000
D = 128
NC = 2
NS = 16
CHUNK = 80
NCHUNK = 125
R_MAIN = 624
R_LAST = N_NODES - (NS - 1) * R_MAIN


def _mm_body(x_ref, w_ref, y_ref):
    y_ref[...] = jnp.dot(x_ref[...], w_ref[...],
                         preferred_element_type=jnp.float32)


def _matmul(x, w, blk=2000):
    n = x.shape[0]
    return pl.pallas_call(
        _mm_body,
        grid=(n // blk,),
        in_specs=[pl.BlockSpec((blk, D), lambda i: (i, 0)),
                  pl.BlockSpec((D, D), lambda i: (0, 0))],
        out_specs=pl.BlockSpec((blk, D), lambda i: (i, 0)),
        out_shape=jax.ShapeDtypeStruct((n, D), jnp.float32),
    )(x, w)


def _fuse_body(p0_ref, p1_ref, y_ref, b_ref, w_ref, out_ref):
    h = jnp.tanh(p0_ref[...] + p1_ref[...] - y_ref[...] + b_ref[...])
    out_ref[...] = jnp.dot(h, w_ref[...], preferred_element_type=jnp.float32)


def _fused_tanh_matmul(p0, p1, y, b, w, blk=2000):
    n = y.shape[0]
    row = pl.BlockSpec((blk, D), lambda i: (i, 0))
    return pl.pallas_call(
        _fuse_body,
        grid=(n // blk,),
        in_specs=[row, row, row,
                  pl.BlockSpec((1, D), lambda i: (0, 0)),
                  pl.BlockSpec((D, D), lambda i: (0, 0))],
        out_specs=row,
        out_shape=jax.ShapeDtypeStruct((n, D), jnp.float32),
    )(p0, p1, y, b.reshape(1, D), w)


def _final_body(q0_ref, q1_ref, y_ref, b_ref, wo_ref, bo_ref, out_ref, acc):
    i = pl.program_id(0)
    h = jnp.tanh(q0_ref[...] + q1_ref[...] - y_ref[...] + b_ref[...])
    s = jnp.sum(h, axis=0, keepdims=True)

    @pl.when(i == 0)
    def _():
        acc[...] = jnp.broadcast_to(s, acc.shape)

    @pl.when(i > 0)
    def _():
        acc[...] += jnp.broadcast_to(s, acc.shape)

    @pl.when(i == pl.num_programs(0) - 1)
    def _():
        mean = acc[...][0:1, :] * (1.0 / N_NODES)
        out_ref[...] = jnp.tanh(
            jnp.dot(mean, wo_ref[...], preferred_element_type=jnp.float32)
            + bo_ref[...])


def _final_readout(q0, q1, y, b, wo_pad, bo_pad, blk=2000):
    n = y.shape[0]
    row = pl.BlockSpec((blk, D), lambda i: (i, 0))
    one = pl.BlockSpec((1, D), lambda i: (0, 0))
    return pl.pallas_call(
        _final_body,
        grid=(n // blk,),
        in_specs=[row, row, row, one,
                  pl.BlockSpec((D, D), lambda i: (0, 0)), one],
        out_specs=one,
        out_shape=jax.ShapeDtypeStruct((1, D), jnp.float32),
        scratch_shapes=[pltpu.VMEM((8, D), jnp.float32)],
    )(q0, q1, y, b.reshape(1, D), wo_pad, bo_pad)


def _sc_agg(y, ei):
    mesh = plsc.VectorSubcoreMesh(core_axis_name="c", subcore_axis_name="s")

    @functools.partial(
        pl.kernel,
        out_type=[jax.ShapeDtypeStruct((N_NODES, D), jnp.float32),
                  jax.ShapeDtypeStruct((N_NODES, D), jnp.float32)],
        mesh=mesh,
        scratch_types=[
            pltpu.VMEM((NCHUNK, CHUNK), jnp.int32),
            pltpu.VMEM((NCHUNK, CHUNK), jnp.int32),
            pltpu.VMEM((CHUNK, D), jnp.float32),
            pltpu.VMEM_SHARED((N_NODES, D), jnp.float32),
            pltpu.SemaphoreType.DMA,
        ],
    )
    def k(y_hbm, ei_hbm, p0_hbm, p1_hbm, src_v, dst_v, rows_v, acc_sh, sem):
        c = lax.axis_index("c")
        s = lax.axis_index("s")
        r0 = s * R_MAIN

        @pl.when(s < NS - 1)
        def _():
            pltpu.sync_copy(y_hbm.at[pl.ds(r0, R_MAIN)],
                            acc_sh.at[pl.ds(r0, R_MAIN)])

        @pl.when(s == NS - 1)
        def _():
            pltpu.sync_copy(y_hbm.at[pl.ds((NS - 1) * R_MAIN, R_LAST)],
                            acc_sh.at[pl.ds((NS - 1) * R_MAIN, R_LAST)])

        pltpu.sync_copy(ei_hbm.at[0, c, s], src_v)
        pltpu.sync_copy(ei_hbm.at[1, c, s], dst_v)
        plsc.subcore_barrier()

        def body(j, carry):
            pltpu.async_copy(y_hbm.at[src_v.at[j]], rows_v, sem).wait()
            pltpu.sync_copy(rows_v, acc_sh.at[dst_v.at[j]], add=True)
            return carry

        lax.fori_loop(0, NCHUNK, body, 0)
        plsc.subcore_barrier()

        out_hbm = [p0_hbm, p1_hbm]
        for cc in (0, 1):
            @pl.when((c == cc) & (s < NS - 1))
            def _(cc=cc):
                pltpu.sync_copy(acc_sh.at[pl.ds(r0, R_MAIN)],
                                out_hbm[cc].at[pl.ds(r0, R_MAIN)])

            @pl.when((c == cc) & (s == NS - 1))
            def _(cc=cc):
                pltpu.sync_copy(
                    acc_sh.at[pl.ds((NS - 1) * R_MAIN, R_LAST)],
                    out_hbm[cc].at[pl.ds((NS - 1) * R_MAIN, R_LAST)])

    return k(y, ei)


def kernel(x, edge_index, pos, W1, b1, W2, b2, Wout, bout):
    del pos
    ei = edge_index.astype(jnp.int32).reshape(2, NC, NS, NCHUNK, CHUNK)

    y1 = _matmul(x, W1)
    p0, p1 = _sc_agg(y1, ei)
    y2 = _fused_tanh_matmul(p0, p1, y1, b1, W2)
    q0, q1 = _sc_agg(y2, ei)

    wo_pad = jnp.zeros((D, D), jnp.float32).at[:, :2].set(Wout)
    bo_pad = jnp.zeros((1, D), jnp.float32).at[0, :2].set(bout)
    out_pad = _final_readout(q0, q1, y2, b2, wo_pad, bo_pad)
    return out_pad[0, :2]
